# baseline (speedup 1.0000x reference)
import dataclasses
import functools

import jax
import jax.numpy as jnp
from jax import lax
from jax.experimental import pallas as pl
from jax.experimental.pallas import tpu as pltpu
from jax.experimental.pallas import tpu_sc as plsc

N_NODES = 10000
D_FEAT = 128
N_EDGES = 320000

NUM_CORES = 2
NUM_SUBCORES = 16
NUM_WORKERS = NUM_CORES * NUM_SUBCORES
LANES = 16

ROWS_PER_WORKER = 320
N_PAD = NUM_WORKERS * ROWS_PER_WORKER
CHUNK = 3200
N_CHUNKS = N_EDGES // CHUNK
GATHER = 128
FCHUNKS = D_FEAT // LANES
BLANES = 2 * LANES
BCHUNKS = D_FEAT // BLANES
DWORDS = D_FEAT // 2
MBUF = CHUNK + 2 * GATHER


def _body(x_hbm, src_hbm, dst_hbm, out_hbm,
          xs, acc, src_a, dst_a, src_b, dst_b,
          msrc_a, mdst_a, msrc_b, mdst_b,
          rows_a, rows_b, rows_c0, rows_c1,
          sem_ca, sem_cb, sem_ga, sem_gb, sem_g0, sem_g1):
    wid = lax.axis_index("c") * NUM_SUBCORES + lax.axis_index("s")
    lo = wid * ROWS_PER_WORKER
    neg = jnp.full((BLANES,), -jnp.inf, jnp.bfloat16)
    padv = jnp.full((LANES,), 0, jnp.int32) + lo
    trashv = jnp.full((LANES,), ROWS_PER_WORKER, jnp.int32)

    @pl.when(lax.axis_index("s") == 0)
    def _():
        pltpu.sync_copy(x_hbm, xs)

    @pl.loop(0, ROWS_PER_WORKER)
    def _(i):
        for j in range(BCHUNKS):
            acc[i, pl.ds(j * BLANES, BLANES)] = neg

    plsc.subcore_barrier()

    def issue_chunk(ci, sbuf, dbuf, sem):
        base = ci * CHUNK
        pltpu.async_copy(src_hbm.at[pl.ds(base, CHUNK)], sbuf, sem)
        pltpu.async_copy(dst_hbm.at[pl.ds(base, CHUNK)], dbuf, sem)

    def wait_chunk(sbuf, dbuf, sem):
        pltpu.make_async_copy(src_hbm.at[pl.ds(0, CHUNK)], sbuf, sem).wait()
        pltpu.make_async_copy(dst_hbm.at[pl.ds(0, CHUNK)], dbuf, sem).wait()

    def issue_gather(g, msrc, buf, sem):
        pltpu.async_copy(xs.at[msrc.at[pl.ds(g * GATHER, GATHER)]],
                         buf, sem)

    def wait_gather(msrc, buf, sem):
        pltpu.make_async_copy(xs.at[msrc.at[pl.ds(0, GATHER)]],
                              buf, sem).wait()

    def scan_chunk(src_c, dst_c, msrc, mdst):
        @functools.partial(
            plsc.parallel_loop(0, CHUNK // LANES, unroll=8,
                               carry=jnp.zeros((LANES,), jnp.int32)))
        def offv(i, off):
            s = src_c[pl.ds(i * LANES, LANES)]
            d = dst_c[pl.ds(i * LANES, LANES)]
            m = (d >= lo) & (d < lo + ROWS_PER_WORKER)
            mi = m.astype(jnp.int32)
            pos = off + plsc.cumsum(mi) - 1
            plsc.store_scatter(msrc, [pos], s, mask=m)
            plsc.store_scatter(mdst, [pos], d - lo, mask=m)
            return off + plsc.all_reduce_population_count(m)

        k = jnp.max(offv)

        iot = lax.iota(jnp.int32, LANES)
        kal = (k // LANES) * LANES
        for j in range(FCHUNKS + 3):
            pos = kal + j * LANES + iot
            m = pos >= k
            plsc.store_scatter(msrc, [pos], padv, mask=m)
            plsc.store_scatter(mdst, [pos], trashv, mask=m)
        return k

    def apply_batch(k, g, rows, mdst):
        gbase = g * GATHER
        rcnt = jnp.minimum(GATHER, k - gbase)
        nquad = (rcnt + 3) // 4

        def apply_one(r, ld):
            avs = [acc[ld, pl.ds(j * BLANES, BLANES)] for j in range(BCHUNKS)]
            rvs = [plsc.bitcast(rows[r, pl.ds(j * LANES, LANES)],
                                jnp.bfloat16) for j in range(BCHUNKS)]
            for j in range(BCHUNKS):
                acc[ld, pl.ds(j * BLANES, BLANES)] = jnp.maximum(avs[j], rvs[j])

        def apply_body(p, lds):
            r0 = 4 * p
            nxt = tuple(mdst[pl.ds(gbase + r0 + 4 + i, LANES)][0]
                        for i in range(4))
            for i in range(4):
                apply_one(r0 + i, lds[i])
            return nxt

        lds0 = tuple(mdst[pl.ds(gbase + i, LANES)][0] for i in range(4))
        lax.fori_loop(0, nquad, apply_body, lds0)

    def apply_chunk(k, msrc, mdst, rows_c, sem_c):
        nb = (k + GATHER - 1) // GATHER
        wait_gather(msrc, rows_c, sem_c)

        @pl.when(nb > 1)
        def _():
            issue_gather(1, msrc, rows_a, sem_ga)

        @pl.when(nb > 0)
        def _():
            apply_batch(k, 0, rows_c, mdst)

        def pair_body(h, _):
            g0 = 1 + 2 * h
            g1 = g0 + 1
            wait_gather(msrc, rows_a, sem_ga)

            @pl.when(g1 < nb)
            def _():
                issue_gather(g1, msrc, rows_b, sem_gb)

            apply_batch(k, g0, rows_a, mdst)

            @pl.when(g1 < nb)
            def _():
                wait_gather(msrc, rows_b, sem_gb)

                @pl.when(g1 + 1 < nb)
                def _():
                    issue_gather(g1 + 1, msrc, rows_a, sem_ga)

                apply_batch(k, g1, rows_b, mdst)

            return 0

        lax.fori_loop(0, nb // 2, pair_body, 0)

    issue_chunk(0, src_a, dst_a, sem_ca)
    wait_chunk(src_a, dst_a, sem_ca)
    issue_chunk(1, src_b, dst_b, sem_cb)
    k0 = scan_chunk(src_a, dst_a, msrc_a, mdst_a)
    issue_gather(0, msrc_a, rows_c0, sem_g0)

    def pipe_body(cp, k_even):
        wait_chunk(src_b, dst_b, sem_cb)
        k_odd = scan_chunk(src_b, dst_b, msrc_b, mdst_b)
        issue_gather(0, msrc_b, rows_c1, sem_g1)
        issue_chunk(2 * cp + 2, src_a, dst_a, sem_ca)
        apply_chunk(k_even, msrc_a, mdst_a, rows_c0, sem_g0)

        wait_chunk(src_a, dst_a, sem_ca)
        k_even = scan_chunk(src_a, dst_a, msrc_a, mdst_a)
        issue_gather(0, msrc_a, rows_c0, sem_g0)
        issue_chunk(2 * cp + 3, src_b, dst_b, sem_cb)
        apply_chunk(k_odd, msrc_b, mdst_b, rows_c1, sem_g1)
        return k_even

    k_last = lax.fori_loop(0, N_CHUNKS // 2 - 1, pipe_body, k0)

    wait_chunk(src_b, dst_b, sem_cb)
    k49 = scan_chunk(src_b, dst_b, msrc_b, mdst_b)
    issue_gather(0, msrc_b, rows_c1, sem_g1)
    apply_chunk(k_last, msrc_a, mdst_a, rows_c0, sem_g0)
    apply_chunk(k49, msrc_b, mdst_b, rows_c1, sem_g1)

    @pl.loop(0, ROWS_PER_WORKER)
    def _(i):
        for j in range(BCHUNKS):
            sl = pl.ds(j * BLANES, BLANES)
            v = acc[i, sl]
            acc[i, sl] = jnp.where(v == -jnp.inf,
                                   jnp.zeros((BLANES,), jnp.bfloat16), v)

    pltpu.sync_copy(acc.at[pl.ds(0, ROWS_PER_WORKER)],
                    out_hbm.at[pl.ds(lo, ROWS_PER_WORKER)])


@jax.jit
def kernel(x, edge_index):
    src = edge_index[0]
    dst = edge_index[1]
    xb = x.astype(jnp.bfloat16)
    x32 = lax.bitcast_convert_type(
        xb.reshape(N_NODES, D_FEAT // 2, 2), jnp.int32)

    cp = pltpu.CompilerParams()
    if "needs_layout_passes" in pltpu.CompilerParams.__dataclass_fields__:
        cp = dataclasses.replace(cp, needs_layout_passes=False)
    cp = dataclasses.replace(cp, use_tc_tiling_on_sc=False)

    mesh = plsc.VectorSubcoreMesh(core_axis_name="c", subcore_axis_name="s")
    run = pl.kernel(
        _body,
        out_type=jax.ShapeDtypeStruct((N_PAD, D_FEAT), jnp.bfloat16),
        mesh=mesh,
        scratch_types=[
            pltpu.VMEM_SHARED((N_NODES, DWORDS), jnp.int32),
            pltpu.VMEM((ROWS_PER_WORKER + 1, D_FEAT), jnp.bfloat16),
            pltpu.VMEM((CHUNK,), jnp.int32),
            pltpu.VMEM((CHUNK,), jnp.int32),
            pltpu.VMEM((CHUNK,), jnp.int32),
            pltpu.VMEM((CHUNK,), jnp.int32),
            pltpu.VMEM((MBUF,), jnp.int32),
            pltpu.VMEM((MBUF,), jnp.int32),
            pltpu.VMEM((MBUF,), jnp.int32),
            pltpu.VMEM((MBUF,), jnp.int32),
            pltpu.VMEM((GATHER, DWORDS), jnp.int32),
            pltpu.VMEM((GATHER, DWORDS), jnp.int32),
            pltpu.VMEM((GATHER, DWORDS), jnp.int32),
            pltpu.VMEM((GATHER, DWORDS), jnp.int32),
            pltpu.SemaphoreType.DMA,
            pltpu.SemaphoreType.DMA,
            pltpu.SemaphoreType.DMA,
            pltpu.SemaphoreType.DMA,
            pltpu.SemaphoreType.DMA,
            pltpu.SemaphoreType.DMA,
        ],
        compiler_params=cp,
    )
    out = run(x32, src, dst)
    return out[:N_NODES].astype(jnp.float32)


# speedup vs baseline: 1.0835x; 1.0835x over previous
import dataclasses
import functools

import jax
import jax.numpy as jnp
from jax import lax
from jax.experimental import pallas as pl
from jax.experimental.pallas import tpu as pltpu
from jax.experimental.pallas import tpu_sc as plsc

N_NODES = 10000
D_FEAT = 128
N_EDGES = 320000

NUM_CORES = 2
NUM_SUBCORES = 16
NUM_WORKERS = NUM_CORES * NUM_SUBCORES
LANES = 16

ROWS_PER_WORKER = 320
N_PAD = NUM_WORKERS * ROWS_PER_WORKER
CHUNK = 4000
N_CHUNKS = N_EDGES // CHUNK
GATHER = 128
FCHUNKS = D_FEAT // LANES
BLANES = 2 * LANES
BCHUNKS = D_FEAT // BLANES
DWORDS = D_FEAT // 2
MBUF = CHUNK + 2 * GATHER


def _body(x_hbm, src_hbm, dst_hbm, out_hbm,
          xs, acc, src_a, dst_a, src_b, dst_b,
          msrc_a, mdst_a, msrc_b, mdst_b,
          rows_a, rows_b, rows_c0, rows_c1,
          sem_ca, sem_cb, sem_ga, sem_gb, sem_g0, sem_g1):
    wid = lax.axis_index("c") * NUM_SUBCORES + lax.axis_index("s")
    lo = wid * ROWS_PER_WORKER
    neg = jnp.full((BLANES,), -jnp.inf, jnp.bfloat16)
    padv = jnp.full((LANES,), 0, jnp.int32) + lo
    trashv = jnp.full((LANES,), ROWS_PER_WORKER, jnp.int32)

    @pl.when(lax.axis_index("s") == 0)
    def _():
        pltpu.sync_copy(x_hbm, xs)

    @pl.loop(0, ROWS_PER_WORKER)
    def _(i):
        for j in range(BCHUNKS):
            acc[i, pl.ds(j * BLANES, BLANES)] = neg

    plsc.subcore_barrier()

    def issue_chunk(ci, sbuf, dbuf, sem):
        base = ci * CHUNK
        pltpu.async_copy(src_hbm.at[pl.ds(base, CHUNK)], sbuf, sem)
        pltpu.async_copy(dst_hbm.at[pl.ds(base, CHUNK)], dbuf, sem)

    def wait_chunk(sbuf, dbuf, sem):
        pltpu.make_async_copy(src_hbm.at[pl.ds(0, CHUNK)], sbuf, sem).wait()
        pltpu.make_async_copy(dst_hbm.at[pl.ds(0, CHUNK)], dbuf, sem).wait()

    def issue_gather(g, msrc, buf, sem):
        pltpu.async_copy(xs.at[msrc.at[pl.ds(g * GATHER, GATHER)]],
                         buf, sem)

    def wait_gather(msrc, buf, sem):
        pltpu.make_async_copy(xs.at[msrc.at[pl.ds(0, GATHER)]],
                              buf, sem).wait()

    def scan_chunk(src_c, dst_c, msrc, mdst):
        @functools.partial(
            plsc.parallel_loop(0, CHUNK // LANES, unroll=8,
                               carry=jnp.zeros((LANES,), jnp.int32)))
        def offv(i, off):
            s = src_c[pl.ds(i * LANES, LANES)]
            d = dst_c[pl.ds(i * LANES, LANES)]
            m = (d >= lo) & (d < lo + ROWS_PER_WORKER)
            mi = m.astype(jnp.int32)
            pos = off + plsc.cumsum(mi) - 1
            plsc.store_scatter(msrc, [pos], s, mask=m)
            plsc.store_scatter(mdst, [pos], d - lo, mask=m)
            return off + plsc.all_reduce_population_count(m)

        k = jnp.max(offv)

        iot = lax.iota(jnp.int32, LANES)
        kal = (k // LANES) * LANES
        for j in range(FCHUNKS + 1):
            pos = kal + j * LANES + iot
            m = pos >= k
            plsc.store_scatter(msrc, [pos], padv, mask=m)
            plsc.store_scatter(mdst, [pos], trashv, mask=m)
        return k

    def apply_batch(k, g, rows, mdst):
        gbase = g * GATHER
        rcnt = jnp.minimum(GATHER, k - gbase)
        npair = (rcnt + 1) // 2

        def apply_one(r, ld):
            avs = [acc[ld, pl.ds(j * BLANES, BLANES)] for j in range(BCHUNKS)]
            rvs = [plsc.bitcast(rows[r, pl.ds(j * LANES, LANES)],
                                jnp.bfloat16) for j in range(BCHUNKS)]
            for j in range(BCHUNKS):
                acc[ld, pl.ds(j * BLANES, BLANES)] = jnp.maximum(avs[j], rvs[j])

        def apply_body(p, lds):
            ld0, ld1 = lds
            r0 = 2 * p
            ld2 = mdst[pl.ds(gbase + r0 + 2, LANES)][0]
            ld3 = mdst[pl.ds(gbase + r0 + 3, LANES)][0]
            apply_one(r0, ld0)
            apply_one(r0 + 1, ld1)
            return (ld2, ld3)

        ld0 = mdst[pl.ds(gbase, LANES)][0]
        ld1 = mdst[pl.ds(gbase + 1, LANES)][0]
        lax.fori_loop(0, npair, apply_body, (ld0, ld1))

    def apply_chunk(k, msrc, mdst, rows_c, sem_c):
        nb = (k + GATHER - 1) // GATHER
        wait_gather(msrc, rows_c, sem_c)

        @pl.when(nb > 1)
        def _():
            issue_gather(1, msrc, rows_a, sem_ga)

        @pl.when(nb > 0)
        def _():
            apply_batch(k, 0, rows_c, mdst)

        def pair_body(h, _):
            g0 = 1 + 2 * h
            g1 = g0 + 1
            wait_gather(msrc, rows_a, sem_ga)

            @pl.when(g1 < nb)
            def _():
                issue_gather(g1, msrc, rows_b, sem_gb)

            apply_batch(k, g0, rows_a, mdst)

            @pl.when(g1 < nb)
            def _():
                wait_gather(msrc, rows_b, sem_gb)

                @pl.when(g1 + 1 < nb)
                def _():
                    issue_gather(g1 + 1, msrc, rows_a, sem_ga)

                apply_batch(k, g1, rows_b, mdst)

            return 0

        lax.fori_loop(0, nb // 2, pair_body, 0)

    issue_chunk(0, src_a, dst_a, sem_ca)
    wait_chunk(src_a, dst_a, sem_ca)
    issue_chunk(1, src_b, dst_b, sem_cb)
    k0 = scan_chunk(src_a, dst_a, msrc_a, mdst_a)
    issue_gather(0, msrc_a, rows_c0, sem_g0)

    def pipe_body(cp, k_even):
        wait_chunk(src_b, dst_b, sem_cb)
        k_odd = scan_chunk(src_b, dst_b, msrc_b, mdst_b)
        issue_gather(0, msrc_b, rows_c1, sem_g1)
        issue_chunk(2 * cp + 2, src_a, dst_a, sem_ca)
        apply_chunk(k_even, msrc_a, mdst_a, rows_c0, sem_g0)

        wait_chunk(src_a, dst_a, sem_ca)
        k_even = scan_chunk(src_a, dst_a, msrc_a, mdst_a)
        issue_gather(0, msrc_a, rows_c0, sem_g0)
        issue_chunk(2 * cp + 3, src_b, dst_b, sem_cb)
        apply_chunk(k_odd, msrc_b, mdst_b, rows_c1, sem_g1)
        return k_even

    k_last = lax.fori_loop(0, N_CHUNKS // 2 - 1, pipe_body, k0)

    wait_chunk(src_b, dst_b, sem_cb)
    k49 = scan_chunk(src_b, dst_b, msrc_b, mdst_b)
    issue_gather(0, msrc_b, rows_c1, sem_g1)
    apply_chunk(k_last, msrc_a, mdst_a, rows_c0, sem_g0)
    apply_chunk(k49, msrc_b, mdst_b, rows_c1, sem_g1)

    @pl.loop(0, ROWS_PER_WORKER)
    def _(i):
        for j in range(BCHUNKS):
            sl = pl.ds(j * BLANES, BLANES)
            v = acc[i, sl]
            acc[i, sl] = jnp.where(v == -jnp.inf,
                                   jnp.zeros((BLANES,), jnp.bfloat16), v)

    pltpu.sync_copy(acc.at[pl.ds(0, ROWS_PER_WORKER)],
                    out_hbm.at[pl.ds(lo, ROWS_PER_WORKER)])


@jax.jit
def kernel(x, edge_index):
    src = edge_index[0]
    dst = edge_index[1]
    xb = x.astype(jnp.bfloat16)
    x32 = lax.bitcast_convert_type(
        xb.reshape(N_NODES, D_FEAT // 2, 2), jnp.int32)

    cp = pltpu.CompilerParams()
    if "needs_layout_passes" in pltpu.CompilerParams.__dataclass_fields__:
        cp = dataclasses.replace(cp, needs_layout_passes=False)
    cp = dataclasses.replace(cp, use_tc_tiling_on_sc=False)

    mesh = plsc.VectorSubcoreMesh(core_axis_name="c", subcore_axis_name="s")
    run = pl.kernel(
        _body,
        out_type=jax.ShapeDtypeStruct((N_PAD, D_FEAT), jnp.bfloat16),
        mesh=mesh,
        scratch_types=[
            pltpu.VMEM_SHARED((N_NODES, DWORDS), jnp.int32),
            pltpu.VMEM((ROWS_PER_WORKER + 1, D_FEAT), jnp.bfloat16),
            pltpu.VMEM((CHUNK,), jnp.int32),
            pltpu.VMEM((CHUNK,), jnp.int32),
            pltpu.VMEM((CHUNK,), jnp.int32),
            pltpu.VMEM((CHUNK,), jnp.int32),
            pltpu.VMEM((MBUF,), jnp.int32),
            pltpu.VMEM((MBUF,), jnp.int32),
            pltpu.VMEM((MBUF,), jnp.int32),
            pltpu.VMEM((MBUF,), jnp.int32),
            pltpu.VMEM((GATHER, DWORDS), jnp.int32),
            pltpu.VMEM((GATHER, DWORDS), jnp.int32),
            pltpu.VMEM((GATHER, DWORDS), jnp.int32),
            pltpu.VMEM((GATHER, DWORDS), jnp.int32),
            pltpu.SemaphoreType.DMA,
            pltpu.SemaphoreType.DMA,
            pltpu.SemaphoreType.DMA,
            pltpu.SemaphoreType.DMA,
            pltpu.SemaphoreType.DMA,
            pltpu.SemaphoreType.DMA,
        ],
        compiler_params=cp,
    )
    out = run(x32, src, dst)
    return out[:N_NODES].astype(jnp.float32)
